# baseline (speedup 1.0000x reference)
import functools

import jax
import jax.numpy as jnp
from jax import lax
from jax.experimental import pallas as pl
from jax.experimental.pallas import tpu as pltpu
from jax.experimental.pallas import tpu_sc as plsc

NC = 2
NS = 16
NW = NC * NS
B = 16384
V = 100000
D = 32
NSLAB = 25
VPW = NSLAB * 128
SEG = 4096
LANES = 16
FLUSH = 64
CAP = 80


def _emb_kernel(idx_hbm, tt_hbm, out_hbm, buf, iv, lbuf, bbuf, orows, didx,
                sem, osem):
    wid = lax.axis_index("s") * NC + lax.axis_index("c")
    v_lo = wid * VPW
    v_hi = jnp.minimum(v_lo + VPW, V)
    iota = lax.iota(jnp.int32, LANES)
    nslab = (v_hi - v_lo + 127) // 128

    def fire(j, x):
        pltpu.async_copy(tt_hbm.at[:, pl.ds(v_lo + 128 * j, 128)],
                         buf.at[j], sem)
        return x

    lax.fori_loop(0, nslab, fire, 0)

    def drain(j, x):
        pltpu.make_async_copy(tt_hbm.at[:, pl.ds(0, 128)], buf.at[0],
                              sem).wait()
        return x

    lax.fori_loop(0, nslab, drain, 0)

    def flush(nf, n_rows):
        slot = lax.rem(nf, 2)

        @pl.when(nf >= 2)
        def _():
            pltpu.make_async_copy(orows.at[0], out_hbm.at[pl.ds(0, FLUSH)],
                                  osem).wait()

        l0v = lbuf[pl.ds(0, LANES)]
        b0v = bbuf[pl.ds(0, LANES)]
        l0 = jnp.sum(jnp.where(iota == 0, l0v, 0))
        b0 = jnp.sum(jnp.where(iota == 0, b0v, 0))
        zero = jnp.zeros_like(iota)
        for k in range(FLUSH // LANES):
            lk = lbuf[pl.ds(k * LANES, LANES)]
            bk = bbuf[pl.ds(k * LANES, LANES)]
            mt = (k * LANES + iota) < n_rows
            lsel = jnp.where(mt, lk, l0)
            bsel = jnp.where(mt, bk, b0)
            slab16 = lax.shift_right_logical(lsel, 7)
            col16 = lax.bitwise_and(lsel, 127)
            plsc.store_scatter(didx, [slot + zero, k * LANES + iota], bsel)
            for c in range(D):
                col = plsc.load_gather(buf, [slab16, c + zero, col16])
                plsc.store_scatter(
                    orows, [slot + zero, k * LANES + iota, c + zero], col)
        pltpu.async_copy(orows.at[slot], out_hbm.at[didx.at[slot]], osem)

    def seg_body(s, carry):
        hc0, nf0 = carry
        pltpu.sync_copy(idx_hbm.at[pl.ds(s * SEG, SEG)], iv)

        UNROLL = 8

        def chunk_body(cho, carry):
            hc, nf = carry
            for sub in range(UNROLL):
                ch = cho * UNROLL + sub
                v16 = iv[pl.ds(ch * LANES, LANES)]
                local = v16 - v_lo
                m = (v16 >= v_lo) & (v16 < v_hi)
                cnt = jnp.sum(jnp.where(m, 1, 0))

                @pl.when(cnt > 0)
                def _(hc=hc, local=local, m=m, ch=ch):
                    plsc.store_compressed(lbuf.at[pl.ds(hc, LANES)], local,
                                          mask=m)
                    plsc.store_compressed(bbuf.at[pl.ds(hc, LANES)],
                                          s * SEG + ch * LANES + iota,
                                          mask=m)

                hc = hc + cnt

                @pl.when(hc >= FLUSH)
                def _(hc=hc, nf=nf):
                    flush(nf, FLUSH)
                    lt = lbuf[pl.ds(FLUSH, LANES)]
                    bt = bbuf[pl.ds(FLUSH, LANES)]
                    lbuf[pl.ds(0, LANES)] = lt
                    bbuf[pl.ds(0, LANES)] = bt

                nf = nf + jnp.where(hc >= FLUSH, 1, 0)
                hc = jnp.where(hc >= FLUSH, hc - FLUSH, hc)
            return hc, nf

        return lax.fori_loop(0, SEG // LANES // UNROLL, chunk_body,
                             (hc0, nf0))

    hc, nf = lax.fori_loop(0, B // SEG, seg_body, (jnp.int32(0), jnp.int32(0)))

    @pl.when(hc > 0)
    def _():
        flush(nf, hc)

    nf = nf + jnp.where(hc > 0, 1, 0)

    @pl.when(nf >= 2)
    def _():
        pltpu.make_async_copy(orows.at[0], out_hbm.at[pl.ds(0, FLUSH)],
                              osem).wait()

    @pl.when(nf >= 1)
    def _():
        pltpu.make_async_copy(orows.at[0], out_hbm.at[pl.ds(0, FLUSH)],
                              osem).wait()


@jax.jit
def kernel(char_indices, table):
    tt = jnp.swapaxes(table, 0, 1)
    mesh = plsc.VectorSubcoreMesh(
        core_axis_name="c", subcore_axis_name="s", num_cores=NC, num_subcores=NS
    )
    k = functools.partial(
        pl.kernel,
        out_type=jax.ShapeDtypeStruct((B, 128), jnp.float32),
        mesh=mesh,
        compiler_params=pltpu.CompilerParams(needs_layout_passes=False),
        scratch_types=[
            pltpu.VMEM((NSLAB, 32, 128), jnp.float32),
            pltpu.VMEM((SEG,), jnp.int32),
            pltpu.VMEM((CAP,), jnp.int32),
            pltpu.VMEM((CAP,), jnp.int32),
            pltpu.VMEM((2, FLUSH, 128), jnp.float32),
            pltpu.VMEM((2, FLUSH), jnp.int32),
            pltpu.SemaphoreType.DMA,
            pltpu.SemaphoreType.DMA,
        ],
    )(_emb_kernel)
    out128 = k(char_indices.astype(jnp.int32), tt)
    return out128[:, :D]


# speedup vs baseline: 1.1376x; 1.1376x over previous
import functools

import jax
import jax.numpy as jnp
from jax import lax
from jax.experimental import pallas as pl
from jax.experimental.pallas import tpu as pltpu
from jax.experimental.pallas import tpu_sc as plsc

NC = 2
NS = 16
NW = NC * NS
B = 16384
V = 100000
D = 32
NSLAB = 25
VPW = NSLAB * 128
SEG = 4096
LANES = 16
FLUSH = 64
CAP = 80


def _emb_kernel(idx_hbm, tt_hbm, out_hbm, buf, iv, lbuf, bbuf, orows, didx,
                sem, osem):
    wid = lax.axis_index("s") * NC + lax.axis_index("c")
    v_lo = wid * VPW
    v_hi = jnp.minimum(v_lo + VPW, V)
    iota = lax.iota(jnp.int32, LANES)
    nslab = (v_hi - v_lo + 127) // 128

    def fire(j, x):
        pltpu.async_copy(tt_hbm.at[:, pl.ds(v_lo + 128 * j, 128)],
                         buf.at[j], sem)
        return x

    lax.fori_loop(0, nslab, fire, 0)

    def drain(j, x):
        pltpu.make_async_copy(tt_hbm.at[:, pl.ds(0, 128)], buf.at[0],
                              sem).wait()
        return x

    lax.fori_loop(0, nslab, drain, 0)

    def flush(nf, n_rows):
        slot = lax.rem(nf, 2)

        @pl.when(nf >= 2)
        def _():
            pltpu.make_async_copy(orows.at[0], out_hbm.at[pl.ds(0, FLUSH)],
                                  osem).wait()

        l0v = lbuf[pl.ds(0, LANES)]
        b0v = bbuf[pl.ds(0, LANES)]
        l0 = jnp.sum(jnp.where(iota == 0, l0v, 0))
        b0 = jnp.sum(jnp.where(iota == 0, b0v, 0))
        zero = jnp.zeros_like(iota)
        for k in range(FLUSH // LANES):
            lk = lbuf[pl.ds(k * LANES, LANES)]
            bk = bbuf[pl.ds(k * LANES, LANES)]
            mt = (k * LANES + iota) < n_rows
            lsel = jnp.where(mt, lk, l0)
            bsel = jnp.where(mt, bk, b0)
            slab16 = lax.shift_right_logical(lsel, 7)
            col16 = lax.bitwise_and(lsel, 127)
            plsc.store_scatter(didx, [slot + zero, k * LANES + iota], bsel)
            for c in range(D):
                col = plsc.load_gather(buf, [slab16, c + zero, col16])
                plsc.store_scatter(
                    orows, [slot + zero, k * LANES + iota, c + zero], col)
        pltpu.async_copy(orows.at[slot], out_hbm.at[didx.at[slot]], osem)

    def seg_body(s, carry):
        hc0, nf0 = carry
        pltpu.sync_copy(idx_hbm.at[pl.ds(s * SEG, SEG)], iv)

        def chunk_body(ch, carry):
            hc, nf = carry
            v16 = iv[pl.ds(ch * LANES, LANES)]
            local = v16 - v_lo
            m = (v16 >= v_lo) & (v16 < v_hi)
            cnt = jnp.sum(jnp.where(m, 1, 0))

            @pl.when(cnt > 0)
            def _():
                plsc.store_compressed(lbuf.at[pl.ds(hc, LANES)], local,
                                      mask=m)
                plsc.store_compressed(bbuf.at[pl.ds(hc, LANES)],
                                      s * SEG + ch * LANES + iota, mask=m)

            hc = hc + cnt

            @pl.when(hc >= FLUSH)
            def _():
                flush(nf, FLUSH)
                lt = lbuf[pl.ds(FLUSH, LANES)]
                bt = bbuf[pl.ds(FLUSH, LANES)]
                lbuf[pl.ds(0, LANES)] = lt
                bbuf[pl.ds(0, LANES)] = bt

            nf = nf + jnp.where(hc >= FLUSH, 1, 0)
            hc = jnp.where(hc >= FLUSH, hc - FLUSH, hc)
            return hc, nf

        return lax.fori_loop(0, SEG // LANES, chunk_body, (hc0, nf0))

    hc, nf = lax.fori_loop(0, B // SEG, seg_body, (jnp.int32(0), jnp.int32(0)))

    @pl.when(hc > 0)
    def _():
        flush(nf, hc)

    nf = nf + jnp.where(hc > 0, 1, 0)

    @pl.when(nf >= 2)
    def _():
        pltpu.make_async_copy(orows.at[0], out_hbm.at[pl.ds(0, FLUSH)],
                              osem).wait()

    @pl.when(nf >= 1)
    def _():
        pltpu.make_async_copy(orows.at[0], out_hbm.at[pl.ds(0, FLUSH)],
                              osem).wait()


@jax.jit
def kernel(char_indices, table):
    tt = jnp.swapaxes(table, 0, 1)
    mesh = plsc.VectorSubcoreMesh(
        core_axis_name="c", subcore_axis_name="s", num_cores=NC, num_subcores=NS
    )
    k = functools.partial(
        pl.kernel,
        out_type=jax.ShapeDtypeStruct((B, 128), jnp.float32),
        mesh=mesh,
        compiler_params=pltpu.CompilerParams(needs_layout_passes=False),
        scratch_types=[
            pltpu.VMEM((NSLAB, 32, 128), jnp.float32),
            pltpu.VMEM((SEG,), jnp.int32),
            pltpu.VMEM((CAP,), jnp.int32),
            pltpu.VMEM((CAP,), jnp.int32),
            pltpu.VMEM((2, FLUSH, 128), jnp.float32),
            pltpu.VMEM((2, FLUSH), jnp.int32),
            pltpu.SemaphoreType.DMA,
            pltpu.SemaphoreType.DMA,
        ],
    )(_emb_kernel)
    out128 = k(char_indices.astype(jnp.int32), tt)
    return out128[:, :D]
